# Initial kernel scaffold; baseline (speedup 1.0000x reference)
import jax
import jax.numpy as jnp
from jax.experimental import pallas as pl


def kernel(inputs, edge_index, W0, b0, W1, b1, W2, b2):
    raise NotImplementedError("write your pallas kernel here")


# speedup vs baseline: 6.1205x; 6.1205x over previous
import functools

import jax
import jax.numpy as jnp
from jax import lax
from jax.experimental import pallas as pl
from jax.experimental.pallas import tpu as pltpu
from jax.experimental.pallas import tpu_sc as plsc

N = 10000
E = 320000
D_IN = 128
D_H = 128
D_OUT = 64

NC = 2
NS = 16
NW = NC * NS
EPW = E // NW
CHUNK = 80
NCHUNK = EPW // CHUNK
RPT = 624
TAIL = N - NS * RPT

_SC_PARAMS = pltpu.CompilerParams(use_tc_tiling_on_sc=False,
                                  needs_layout_passes=False)

def _mesh():
    return plsc.VectorSubcoreMesh(core_axis_name="c", subcore_axis_name="s",
                                  num_cores=NC, num_subcores=NS)


def _build_deg(interpret=False):
    @functools.partial(
        pl.kernel,
        mesh=_mesh(),
        out_type=jax.ShapeDtypeStruct((2, NW, N), jnp.float32),
        scratch_types=[
            pltpu.VMEM((2, EPW), jnp.int32),
            pltpu.VMEM((N,), jnp.float32),
            pltpu.VMEM((N,), jnp.float32),
        ],
        compiler_params=_SC_PARAMS,
        interpret=interpret,
    )
    def deg_kernel(ei_hbm, degp_hbm, ebuf, hsrc, hdst):
        c = lax.axis_index("c")
        s = lax.axis_index("s")
        wid = c * NS + s
        pltpu.sync_copy(ei_hbm.at[:, pl.ds(wid * EPW, EPW)], ebuf)

        def zero(i, carry):
            hsrc[pl.ds(i * 16, 16)] = jnp.zeros((16,), jnp.float32)
            hdst[pl.ds(i * 16, 16)] = jnp.zeros((16,), jnp.float32)
            return carry

        lax.fori_loop(0, N // 16, zero, 0)

        ones = jnp.ones((16,), jnp.float32)

        def count(i, carry):
            sv = ebuf[0, pl.ds(i * 16, 16)]
            dv = ebuf[1, pl.ds(i * 16, 16)]
            plsc.addupdate_scatter(hsrc, [sv], ones)
            plsc.addupdate_scatter(hdst, [dv], ones)
            return carry

        lax.fori_loop(0, EPW // 16, count, 0)

        pltpu.sync_copy(hsrc, degp_hbm.at[0, wid])
        pltpu.sync_copy(hdst, degp_hbm.at[1, wid])

    return deg_kernel


def _build_mp(D, interpret=False):
    @functools.partial(
        pl.kernel,
        mesh=_mesh(),
        out_type=jax.ShapeDtypeStruct((NC, N, D), jnp.float32),
        scratch_types=[
            pltpu.VMEM((2, CHUNK), jnp.int32),
            pltpu.VMEM((CHUNK, D), jnp.float32),
            pltpu.VMEM_SHARED((N, D), jnp.float32),
            pltpu.SemaphoreType.DMA,
        ],
        compiler_params=_SC_PARAMS,
        interpret=interpret,
    )
    def mp_kernel(xs_hbm, ei_hbm, z_hbm, out_hbm, ibuf, msgs, acc, sem):
        c = lax.axis_index("c")
        s = lax.axis_index("s")
        wid = c * NS + s
        base = s * RPT

        pltpu.sync_copy(z_hbm.at[pl.ds(base, RPT)], acc.at[pl.ds(base, RPT)])

        @pl.when(s == NS - 1)
        def _zero_tail():
            pltpu.sync_copy(z_hbm.at[pl.ds(NS * RPT, TAIL)],
                            acc.at[pl.ds(NS * RPT, TAIL)])

        plsc.subcore_barrier()

        ebase = wid * EPW

        def chunk(j, carry):
            pltpu.sync_copy(ei_hbm.at[:, pl.ds(ebase + j * CHUNK, CHUNK)], ibuf)
            pltpu.async_copy(xs_hbm.at[ibuf.at[0]], msgs, sem).wait()
            pltpu.sync_copy(msgs, acc.at[ibuf.at[1]], add=True)
            return carry

        lax.fori_loop(0, NCHUNK, chunk, 0)
        plsc.subcore_barrier()

        pltpu.sync_copy(acc.at[pl.ds(base, RPT)],
                        out_hbm.at[c, pl.ds(base, RPT)])

        @pl.when(s == NS - 1)
        def _copy_tail():
            pltpu.sync_copy(acc.at[pl.ds(NS * RPT, TAIL)],
                            out_hbm.at[c, pl.ds(NS * RPT, TAIL)])

    return mp_kernel


_BN = 2000


def _norm_body(degp_ref, ns_ref, nd_ref):
    deg = jnp.sum(degp_ref[...], axis=1)
    ns = lax.rsqrt(jnp.maximum(deg[0], 1.0))
    nd = lax.rsqrt(jnp.maximum(deg[1], 1.0))
    ns_ref[...] = ns[None, :]
    nd_ref[...] = nd[None, :]


def _build_norm(interpret=False):
    return pl.pallas_call(
        _norm_body,
        grid=(1,),
        in_specs=[pl.BlockSpec((2, NW, N), lambda i: (0, 0, 0))],
        out_specs=[
            pl.BlockSpec((1, N), lambda i: (0, 0)),
            pl.BlockSpec((1, N), lambda i: (0, 0)),
        ],
        out_shape=[
            jax.ShapeDtypeStruct((1, N), jnp.float32),
            jax.ShapeDtypeStruct((1, N), jnp.float32),
        ],
        interpret=interpret,
    )


def _scale_body(x_ref, ns_ref, xs_ref):
    xs_ref[...] = x_ref[...] * ns_ref[...]


def _build_scale(interpret=False):
    return pl.pallas_call(
        _scale_body,
        grid=(N // _BN,),
        in_specs=[
            pl.BlockSpec((_BN, D_IN), lambda i: (i, 0)),
            pl.BlockSpec((_BN, 1), lambda i: (i, 0)),
        ],
        out_specs=pl.BlockSpec((_BN, D_IN), lambda i: (i, 0)),
        out_shape=jax.ShapeDtypeStruct((N, D_IN), jnp.float32),
        interpret=interpret,
    )


def _layer_body(p_ref, nd_ref, ns_ref, w_ref, b_ref, o_ref):
    agg = (p_ref[0] + p_ref[1]) * nd_ref[...]
    h = jnp.dot(agg, w_ref[...], preferred_element_type=jnp.float32,
                precision=lax.Precision.HIGHEST) + b_ref[...]
    o_ref[...] = jnp.maximum(h, 0.0) * ns_ref[...]


def _build_layer(interpret=False):
    return pl.pallas_call(
        _layer_body,
        grid=(N // _BN,),
        in_specs=[
            pl.BlockSpec((NC, _BN, D_H), lambda i: (0, i, 0)),
            pl.BlockSpec((_BN, 1), lambda i: (i, 0)),
            pl.BlockSpec((_BN, 1), lambda i: (i, 0)),
            pl.BlockSpec((D_H, D_H), lambda i: (0, 0)),
            pl.BlockSpec((1, D_H), lambda i: (0, 0)),
        ],
        out_specs=pl.BlockSpec((_BN, D_H), lambda i: (i, 0)),
        out_shape=jax.ShapeDtypeStruct((N, D_H), jnp.float32),
        interpret=interpret,
    )


def _layer2_body(p_ref, nd_ref, ns_ref, w1_ref, b1_ref, w2_ref, o_ref):
    agg = (p_ref[0] + p_ref[1]) * nd_ref[...]
    h = jnp.dot(agg, w1_ref[...], preferred_element_type=jnp.float32,
                precision=lax.Precision.HIGHEST) + b1_ref[...]
    h = jnp.maximum(h, 0.0) * ns_ref[...]
    o_ref[...] = jnp.dot(h, w2_ref[...], preferred_element_type=jnp.float32,
                         precision=lax.Precision.HIGHEST)


def _build_layer2(interpret=False):
    return pl.pallas_call(
        _layer2_body,
        grid=(N // _BN,),
        in_specs=[
            pl.BlockSpec((NC, _BN, D_H), lambda i: (0, i, 0)),
            pl.BlockSpec((_BN, 1), lambda i: (i, 0)),
            pl.BlockSpec((_BN, 1), lambda i: (i, 0)),
            pl.BlockSpec((D_H, D_H), lambda i: (0, 0)),
            pl.BlockSpec((1, D_H), lambda i: (0, 0)),
            pl.BlockSpec((D_H, D_OUT), lambda i: (0, 0)),
        ],
        out_specs=pl.BlockSpec((_BN, D_OUT), lambda i: (i, 0)),
        out_shape=jax.ShapeDtypeStruct((N, D_OUT), jnp.float32),
        interpret=interpret,
    )


def _final_body(p_ref, nd_ref, b_ref, o_ref):
    agg = (p_ref[0] + p_ref[1]) * nd_ref[...]
    o_ref[...] = jax.nn.sigmoid(agg + b_ref[...])


def _build_final(interpret=False):
    return pl.pallas_call(
        _final_body,
        grid=(N // _BN,),
        in_specs=[
            pl.BlockSpec((NC, _BN, D_OUT), lambda i: (0, i, 0)),
            pl.BlockSpec((_BN, 1), lambda i: (i, 0)),
            pl.BlockSpec((1, D_OUT), lambda i: (0, 0)),
        ],
        out_specs=pl.BlockSpec((_BN, D_OUT), lambda i: (i, 0)),
        out_shape=jax.ShapeDtypeStruct((N, D_OUT), jnp.float32),
        interpret=interpret,
    )


def _build(interpret=False):
    deg = _build_deg(interpret)
    mp128 = _build_mp(D_H, interpret)
    mp64 = _build_mp(D_OUT, interpret)
    norm = _build_norm(interpret)
    scale = _build_scale(interpret)
    layer = _build_layer(interpret)
    layer2 = _build_layer2(interpret)
    final = _build_final(interpret)

    def run(inputs, edge_index, W0, b0, W1, b1, W2, b2):
        degp = deg(edge_index)
        ns_r, nd_r = norm(degp)
        ns = ns_r.reshape(N, 1)
        nd = nd_r.reshape(N, 1)
        xs0 = scale(inputs, ns)
        z128 = jnp.zeros((N, D_H), jnp.float32)
        z64 = jnp.zeros((N, D_OUT), jnp.float32)
        p1 = mp128(xs0, edge_index, z128)
        xs1 = layer(p1, nd, ns, W0, b0.reshape(1, -1))
        p2 = mp128(xs1, edge_index, z128)
        xs2 = layer2(p2, nd, ns, W1, b1.reshape(1, -1), W2)
        p3 = mp64(xs2, edge_index, z64)
        return final(p3, nd, b2.reshape(1, -1))

    return run


_cache = {}


def kernel(inputs, edge_index, W0, b0, W1, b1, W2, b2):
    if "run" not in _cache:
        _cache["run"] = _build()
    return _cache["run"](inputs, edge_index, W0, b0, W1, b1, W2, b2)


# speedup vs baseline: 9.8844x; 1.6150x over previous
import functools

import jax
import jax.numpy as jnp
from jax import lax
from jax.experimental import pallas as pl
from jax.experimental.pallas import tpu as pltpu
from jax.experimental.pallas import tpu_sc as plsc

N = 10000
E = 320000
D_IN = 128
D_H = 128
D_OUT = 64

NC = 2
NS = 16
NW = NC * NS
EPW = E // NW
CHUNK = 80
NCHUNK = EPW // CHUNK
RPT = 624
TAIL = N - NS * RPT

_SC_PARAMS = pltpu.CompilerParams(use_tc_tiling_on_sc=False,
                                  needs_layout_passes=False)

def _mesh():
    return plsc.VectorSubcoreMesh(core_axis_name="c", subcore_axis_name="s",
                                  num_cores=NC, num_subcores=NS)


def _build_deg(interpret=False):
    @functools.partial(
        pl.kernel,
        mesh=_mesh(),
        out_type=jax.ShapeDtypeStruct((2, NW, N), jnp.float32),
        scratch_types=[
            pltpu.VMEM((2, EPW), jnp.int32),
            pltpu.VMEM((N,), jnp.float32),
            pltpu.VMEM((N,), jnp.float32),
        ],
        compiler_params=_SC_PARAMS,
        interpret=interpret,
    )
    def deg_kernel(ei_hbm, degp_hbm, ebuf, hsrc, hdst):
        c = lax.axis_index("c")
        s = lax.axis_index("s")
        wid = c * NS + s
        pltpu.sync_copy(ei_hbm.at[:, pl.ds(wid * EPW, EPW)], ebuf)

        def zero(i, carry):
            hsrc[pl.ds(i * 16, 16)] = jnp.zeros((16,), jnp.float32)
            hdst[pl.ds(i * 16, 16)] = jnp.zeros((16,), jnp.float32)
            return carry

        lax.fori_loop(0, N // 16, zero, 0)

        ones = jnp.ones((16,), jnp.float32)

        def count(i, carry):
            sv = ebuf[0, pl.ds(i * 16, 16)]
            dv = ebuf[1, pl.ds(i * 16, 16)]
            plsc.addupdate_scatter(hsrc, [sv], ones)
            plsc.addupdate_scatter(hdst, [dv], ones)
            return carry

        lax.fori_loop(0, EPW // 16, count, 0)

        pltpu.sync_copy(hsrc, degp_hbm.at[0, wid])
        pltpu.sync_copy(hdst, degp_hbm.at[1, wid])

    return deg_kernel


def _build_mp(D, interpret=False):
    @functools.partial(
        pl.kernel,
        mesh=_mesh(),
        out_type=jax.ShapeDtypeStruct((NC, N, D), jnp.float32),
        scratch_types=[
            pltpu.VMEM((2, 2, CHUNK), jnp.int32),
            pltpu.VMEM((2, CHUNK, D), jnp.float32),
            pltpu.VMEM_SHARED((N, D), jnp.float32),
            pltpu.SemaphoreType.DMA((2,)),
            pltpu.SemaphoreType.DMA((2,)),
        ],
        compiler_params=_SC_PARAMS,
        interpret=interpret,
    )
    def mp_kernel(xs_hbm, ei_hbm, z_hbm, out_hbm, ibuf, msgs, acc, gsem, ssem):
        c = lax.axis_index("c")
        s = lax.axis_index("s")
        wid = c * NS + s
        base = s * RPT

        pltpu.sync_copy(z_hbm.at[pl.ds(base, RPT)], acc.at[pl.ds(base, RPT)])

        @pl.when(s == NS - 1)
        def _zero_tail():
            pltpu.sync_copy(z_hbm.at[pl.ds(NS * RPT, TAIL)],
                            acc.at[pl.ds(NS * RPT, TAIL)])

        plsc.subcore_barrier()

        ebase = wid * EPW

        def idx_load(j, b):
            pltpu.sync_copy(ei_hbm.at[:, pl.ds(ebase + j * CHUNK, CHUNK)],
                            ibuf.at[b])

        def gather(j, b):
            return pltpu.async_copy(xs_hbm.at[ibuf.at[b, 0]], msgs.at[b],
                                    gsem.at[b])

        def scatter(b):
            return pltpu.async_copy(msgs.at[b], acc.at[ibuf.at[b, 1]],
                                    ssem.at[b], add=True)

        idx_load(0, 0)
        gd = gather(0, 0)
        sd = None
        for j in range(NCHUNK):
            b = j % 2
            nb = 1 - b
            if j + 1 < NCHUNK:
                if sd is not None:
                    sd.wait()
                idx_load(j + 1, nb)
                gd_next = gather(j + 1, nb)
            else:
                gd_next = None
            gd.wait()
            sd = scatter(b)
            gd = gd_next
        sd.wait()
        plsc.subcore_barrier()

        pltpu.sync_copy(acc.at[pl.ds(base, RPT)],
                        out_hbm.at[c, pl.ds(base, RPT)])

        @pl.when(s == NS - 1)
        def _copy_tail():
            pltpu.sync_copy(acc.at[pl.ds(NS * RPT, TAIL)],
                            out_hbm.at[c, pl.ds(NS * RPT, TAIL)])

    return mp_kernel


_BN = 2000


def _norm_body(degp_ref, ns_ref, nd_ref):
    deg = jnp.sum(degp_ref[...], axis=1)
    ns = lax.rsqrt(jnp.maximum(deg[0], 1.0))
    nd = lax.rsqrt(jnp.maximum(deg[1], 1.0))
    ns_ref[...] = ns[None, :]
    nd_ref[...] = nd[None, :]


def _build_norm(interpret=False):
    return pl.pallas_call(
        _norm_body,
        grid=(1,),
        in_specs=[pl.BlockSpec((2, NW, N), lambda i: (0, 0, 0))],
        out_specs=[
            pl.BlockSpec((1, N), lambda i: (0, 0)),
            pl.BlockSpec((1, N), lambda i: (0, 0)),
        ],
        out_shape=[
            jax.ShapeDtypeStruct((1, N), jnp.float32),
            jax.ShapeDtypeStruct((1, N), jnp.float32),
        ],
        interpret=interpret,
    )


def _scale_body(x_ref, ns_ref, xs_ref):
    xs_ref[...] = x_ref[...] * ns_ref[...]


def _build_scale(interpret=False):
    return pl.pallas_call(
        _scale_body,
        grid=(N // _BN,),
        in_specs=[
            pl.BlockSpec((_BN, D_IN), lambda i: (i, 0)),
            pl.BlockSpec((_BN, 1), lambda i: (i, 0)),
        ],
        out_specs=pl.BlockSpec((_BN, D_IN), lambda i: (i, 0)),
        out_shape=jax.ShapeDtypeStruct((N, D_IN), jnp.float32),
        interpret=interpret,
    )


def _layer_body(p_ref, nd_ref, ns_ref, w_ref, b_ref, o_ref):
    agg = (p_ref[0] + p_ref[1]) * nd_ref[...]
    h = jnp.dot(agg, w_ref[...], preferred_element_type=jnp.float32,
                precision=lax.Precision.HIGHEST) + b_ref[...]
    o_ref[...] = jnp.maximum(h, 0.0) * ns_ref[...]


def _build_layer(interpret=False):
    return pl.pallas_call(
        _layer_body,
        grid=(N // _BN,),
        in_specs=[
            pl.BlockSpec((NC, _BN, D_H), lambda i: (0, i, 0)),
            pl.BlockSpec((_BN, 1), lambda i: (i, 0)),
            pl.BlockSpec((_BN, 1), lambda i: (i, 0)),
            pl.BlockSpec((D_H, D_H), lambda i: (0, 0)),
            pl.BlockSpec((1, D_H), lambda i: (0, 0)),
        ],
        out_specs=pl.BlockSpec((_BN, D_H), lambda i: (i, 0)),
        out_shape=jax.ShapeDtypeStruct((N, D_H), jnp.float32),
        interpret=interpret,
    )


def _layer2_body(p_ref, nd_ref, ns_ref, w1_ref, b1_ref, w2_ref, o_ref):
    agg = (p_ref[0] + p_ref[1]) * nd_ref[...]
    h = jnp.dot(agg, w1_ref[...], preferred_element_type=jnp.float32,
                precision=lax.Precision.HIGHEST) + b1_ref[...]
    h = jnp.maximum(h, 0.0) * ns_ref[...]
    o_ref[...] = jnp.dot(h, w2_ref[...], preferred_element_type=jnp.float32,
                         precision=lax.Precision.HIGHEST)


def _build_layer2(interpret=False):
    return pl.pallas_call(
        _layer2_body,
        grid=(N // _BN,),
        in_specs=[
            pl.BlockSpec((NC, _BN, D_H), lambda i: (0, i, 0)),
            pl.BlockSpec((_BN, 1), lambda i: (i, 0)),
            pl.BlockSpec((_BN, 1), lambda i: (i, 0)),
            pl.BlockSpec((D_H, D_H), lambda i: (0, 0)),
            pl.BlockSpec((1, D_H), lambda i: (0, 0)),
            pl.BlockSpec((D_H, D_OUT), lambda i: (0, 0)),
        ],
        out_specs=pl.BlockSpec((_BN, D_OUT), lambda i: (i, 0)),
        out_shape=jax.ShapeDtypeStruct((N, D_OUT), jnp.float32),
        interpret=interpret,
    )


def _final_body(p_ref, nd_ref, b_ref, o_ref):
    agg = (p_ref[0] + p_ref[1]) * nd_ref[...]
    o_ref[...] = jax.nn.sigmoid(agg + b_ref[...])


def _build_final(interpret=False):
    return pl.pallas_call(
        _final_body,
        grid=(N // _BN,),
        in_specs=[
            pl.BlockSpec((NC, _BN, D_OUT), lambda i: (0, i, 0)),
            pl.BlockSpec((_BN, 1), lambda i: (i, 0)),
            pl.BlockSpec((1, D_OUT), lambda i: (0, 0)),
        ],
        out_specs=pl.BlockSpec((_BN, D_OUT), lambda i: (i, 0)),
        out_shape=jax.ShapeDtypeStruct((N, D_OUT), jnp.float32),
        interpret=interpret,
    )


def _build(interpret=False):
    deg = _build_deg(interpret)
    mp128 = _build_mp(D_H, interpret)
    mp64 = _build_mp(D_OUT, interpret)
    norm = _build_norm(interpret)
    scale = _build_scale(interpret)
    layer = _build_layer(interpret)
    layer2 = _build_layer2(interpret)
    final = _build_final(interpret)

    def run(inputs, edge_index, W0, b0, W1, b1, W2, b2):
        degp = deg(edge_index)
        ns_r, nd_r = norm(degp)
        ns = ns_r.reshape(N, 1)
        nd = nd_r.reshape(N, 1)
        xs0 = scale(inputs, ns)
        z128 = jnp.zeros((N, D_H), jnp.float32)
        z64 = jnp.zeros((N, D_OUT), jnp.float32)
        p1 = mp128(xs0, edge_index, z128)
        xs1 = layer(p1, nd, ns, W0, b0.reshape(1, -1))
        p2 = mp128(xs1, edge_index, z128)
        xs2 = layer2(p2, nd, ns, W1, b1.reshape(1, -1), W2)
        p3 = mp64(xs2, edge_index, z64)
        return final(p3, nd, b2.reshape(1, -1))

    return run


_cache = {}


def kernel(inputs, edge_index, W0, b0, W1, b1, W2, b2):
    if "run" not in _cache:
        _cache["run"] = _build()
    return _cache["run"](inputs, edge_index, W0, b0, W1, b1, W2, b2)


# speedup vs baseline: 13.4843x; 1.3642x over previous
import functools

import jax
import jax.numpy as jnp
from jax import lax
from jax.experimental import pallas as pl
from jax.experimental.pallas import tpu as pltpu
from jax.experimental.pallas import tpu_sc as plsc

N = 10000
E = 320000
D_IN = 128
D_H = 128
D_OUT = 64

NC = 2
NS = 16
NW = NC * NS
EPW = E // NW
CHUNK = 80
NCHUNK = EPW // CHUNK
RPT = 624
TAIL = N - NS * RPT

_SC_PARAMS = pltpu.CompilerParams(use_tc_tiling_on_sc=False,
                                  needs_layout_passes=False)

def _mesh():
    return plsc.VectorSubcoreMesh(core_axis_name="c", subcore_axis_name="s",
                                  num_cores=NC, num_subcores=NS)


def _build_deg(interpret=False):
    @functools.partial(
        pl.kernel,
        mesh=_mesh(),
        out_type=jax.ShapeDtypeStruct((2, NW, N), jnp.float32),
        scratch_types=[
            pltpu.VMEM((2, EPW), jnp.int32),
            pltpu.VMEM((N,), jnp.float32),
            pltpu.VMEM((N,), jnp.float32),
        ],
        compiler_params=_SC_PARAMS,
        interpret=interpret,
    )
    def deg_kernel(ei_hbm, degp_hbm, ebuf, hsrc, hdst):
        c = lax.axis_index("c")
        s = lax.axis_index("s")
        wid = c * NS + s
        pltpu.sync_copy(ei_hbm.at[:, pl.ds(wid * EPW, EPW)], ebuf)

        def zero(i, carry):
            hsrc[pl.ds(i * 16, 16)] = jnp.zeros((16,), jnp.float32)
            hdst[pl.ds(i * 16, 16)] = jnp.zeros((16,), jnp.float32)
            return carry

        lax.fori_loop(0, N // 16, zero, 0)

        ones = jnp.ones((16,), jnp.float32)

        def count(i, carry):
            sv = ebuf[0, pl.ds(i * 16, 16)]
            dv = ebuf[1, pl.ds(i * 16, 16)]
            plsc.addupdate_scatter(hsrc, [sv], ones)
            plsc.addupdate_scatter(hdst, [dv], ones)
            return carry

        lax.fori_loop(0, EPW // 16, count, 0)

        pltpu.sync_copy(hsrc, degp_hbm.at[0, wid])
        pltpu.sync_copy(hdst, degp_hbm.at[1, wid])

    return deg_kernel


def _build_mp(D, interpret=False):
    @functools.partial(
        pl.kernel,
        mesh=_mesh(),
        out_type=jax.ShapeDtypeStruct((NC, N, D), jnp.float32),
        scratch_types=[
            pltpu.VMEM((8, 2, CHUNK), jnp.int32),
            pltpu.VMEM((4, CHUNK, D), jnp.float32),
            pltpu.VMEM_SHARED((N, D), jnp.float32),
            pltpu.SemaphoreType.DMA((8,)),
            pltpu.SemaphoreType.DMA((4,)),
            pltpu.SemaphoreType.DMA((4,)),
        ],
        compiler_params=_SC_PARAMS,
        interpret=interpret,
    )
    def mp_kernel(xs_hbm, ei_hbm, z_hbm, out_hbm, ibuf, msgs, acc,
                  isem, gsem, ssem):
        c = lax.axis_index("c")
        s = lax.axis_index("s")
        wid = c * NS + s
        base = s * RPT

        pltpu.sync_copy(z_hbm.at[pl.ds(base, RPT)], acc.at[pl.ds(base, RPT)])

        @pl.when(s == NS - 1)
        def _zero_tail():
            pltpu.sync_copy(z_hbm.at[pl.ds(NS * RPT, TAIL)],
                            acc.at[pl.ds(NS * RPT, TAIL)])

        plsc.subcore_barrier()

        ebase = wid * EPW

        def idx_load(j):
            b = j % 8
            return pltpu.async_copy(
                ei_hbm.at[:, pl.ds(ebase + j * CHUNK, CHUNK)],
                ibuf.at[b], isem.at[b])

        def gather(j):
            return pltpu.async_copy(xs_hbm.at[ibuf.at[j % 8, 0]],
                                    msgs.at[j % 4], gsem.at[j % 4])

        def scatter(j):
            return pltpu.async_copy(msgs.at[j % 4], acc.at[ibuf.at[j % 8, 1]],
                                    ssem.at[j % 4], add=True)

        idesc = {}
        gdesc = {}
        sdesc = {}
        for j in range(min(3, NCHUNK)):
            idesc[j] = idx_load(j)
        for j in range(min(2, NCHUNK)):
            idesc[j].wait()
            gdesc[j] = gather(j)
        for j in range(NCHUNK):
            if j + 2 < NCHUNK and j >= 2:
                sdesc[j - 2].wait()
            if j + 3 < NCHUNK:
                idesc[j + 3] = idx_load(j + 3)
            if j + 2 < NCHUNK:
                idesc[j + 2].wait()
                gdesc[j + 2] = gather(j + 2)
            gdesc[j].wait()
            sdesc[j] = scatter(j)
        for k in range(max(0, NCHUNK - 4), NCHUNK):
            sdesc[k].wait()
        plsc.subcore_barrier()

        pltpu.sync_copy(acc.at[pl.ds(base, RPT)],
                        out_hbm.at[c, pl.ds(base, RPT)])

        @pl.when(s == NS - 1)
        def _copy_tail():
            pltpu.sync_copy(acc.at[pl.ds(NS * RPT, TAIL)],
                            out_hbm.at[c, pl.ds(NS * RPT, TAIL)])

    return mp_kernel


_BN = 2000


def _norm_body(degp_ref, ns_ref, nd_ref):
    deg = jnp.sum(degp_ref[...], axis=1)
    ns = lax.rsqrt(jnp.maximum(deg[0], 1.0))
    nd = lax.rsqrt(jnp.maximum(deg[1], 1.0))
    ns_ref[...] = ns[None, :]
    nd_ref[...] = nd[None, :]


def _build_norm(interpret=False):
    return pl.pallas_call(
        _norm_body,
        grid=(1,),
        in_specs=[pl.BlockSpec((2, NW, N), lambda i: (0, 0, 0))],
        out_specs=[
            pl.BlockSpec((1, N), lambda i: (0, 0)),
            pl.BlockSpec((1, N), lambda i: (0, 0)),
        ],
        out_shape=[
            jax.ShapeDtypeStruct((1, N), jnp.float32),
            jax.ShapeDtypeStruct((1, N), jnp.float32),
        ],
        interpret=interpret,
    )


def _scale_body(x_ref, ns_ref, xs_ref):
    xs_ref[...] = x_ref[...] * ns_ref[...]


def _build_scale(interpret=False):
    return pl.pallas_call(
        _scale_body,
        grid=(N // _BN,),
        in_specs=[
            pl.BlockSpec((_BN, D_IN), lambda i: (i, 0)),
            pl.BlockSpec((_BN, 1), lambda i: (i, 0)),
        ],
        out_specs=pl.BlockSpec((_BN, D_IN), lambda i: (i, 0)),
        out_shape=jax.ShapeDtypeStruct((N, D_IN), jnp.float32),
        interpret=interpret,
    )


def _layer_body(p_ref, nd_ref, ns_ref, w_ref, b_ref, o_ref):
    agg = (p_ref[0] + p_ref[1]) * nd_ref[...]
    h = jnp.dot(agg, w_ref[...], preferred_element_type=jnp.float32,
                precision=lax.Precision.HIGHEST) + b_ref[...]
    o_ref[...] = jnp.maximum(h, 0.0) * ns_ref[...]


def _build_layer(interpret=False):
    return pl.pallas_call(
        _layer_body,
        grid=(N // _BN,),
        in_specs=[
            pl.BlockSpec((NC, _BN, D_H), lambda i: (0, i, 0)),
            pl.BlockSpec((_BN, 1), lambda i: (i, 0)),
            pl.BlockSpec((_BN, 1), lambda i: (i, 0)),
            pl.BlockSpec((D_H, D_H), lambda i: (0, 0)),
            pl.BlockSpec((1, D_H), lambda i: (0, 0)),
        ],
        out_specs=pl.BlockSpec((_BN, D_H), lambda i: (i, 0)),
        out_shape=jax.ShapeDtypeStruct((N, D_H), jnp.float32),
        interpret=interpret,
    )


def _layer2_body(p_ref, nd_ref, ns_ref, w1_ref, b1_ref, w2_ref, o_ref):
    agg = (p_ref[0] + p_ref[1]) * nd_ref[...]
    h = jnp.dot(agg, w1_ref[...], preferred_element_type=jnp.float32,
                precision=lax.Precision.HIGHEST) + b1_ref[...]
    h = jnp.maximum(h, 0.0) * ns_ref[...]
    o_ref[...] = jnp.dot(h, w2_ref[...], preferred_element_type=jnp.float32,
                         precision=lax.Precision.HIGHEST)


def _build_layer2(interpret=False):
    return pl.pallas_call(
        _layer2_body,
        grid=(N // _BN,),
        in_specs=[
            pl.BlockSpec((NC, _BN, D_H), lambda i: (0, i, 0)),
            pl.BlockSpec((_BN, 1), lambda i: (i, 0)),
            pl.BlockSpec((_BN, 1), lambda i: (i, 0)),
            pl.BlockSpec((D_H, D_H), lambda i: (0, 0)),
            pl.BlockSpec((1, D_H), lambda i: (0, 0)),
            pl.BlockSpec((D_H, D_OUT), lambda i: (0, 0)),
        ],
        out_specs=pl.BlockSpec((_BN, D_OUT), lambda i: (i, 0)),
        out_shape=jax.ShapeDtypeStruct((N, D_OUT), jnp.float32),
        interpret=interpret,
    )


def _final_body(p_ref, nd_ref, b_ref, o_ref):
    agg = (p_ref[0] + p_ref[1]) * nd_ref[...]
    o_ref[...] = jax.nn.sigmoid(agg + b_ref[...])


def _build_final(interpret=False):
    return pl.pallas_call(
        _final_body,
        grid=(N // _BN,),
        in_specs=[
            pl.BlockSpec((NC, _BN, D_OUT), lambda i: (0, i, 0)),
            pl.BlockSpec((_BN, 1), lambda i: (i, 0)),
            pl.BlockSpec((1, D_OUT), lambda i: (0, 0)),
        ],
        out_specs=pl.BlockSpec((_BN, D_OUT), lambda i: (i, 0)),
        out_shape=jax.ShapeDtypeStruct((N, D_OUT), jnp.float32),
        interpret=interpret,
    )


def _build(interpret=False):
    deg = _build_deg(interpret)
    mp128 = _build_mp(D_H, interpret)
    mp64 = _build_mp(D_OUT, interpret)
    norm = _build_norm(interpret)
    scale = _build_scale(interpret)
    layer = _build_layer(interpret)
    layer2 = _build_layer2(interpret)
    final = _build_final(interpret)

    def run(inputs, edge_index, W0, b0, W1, b1, W2, b2):
        degp = deg(edge_index)
        ns_r, nd_r = norm(degp)
        ns = ns_r.reshape(N, 1)
        nd = nd_r.reshape(N, 1)
        xs0 = scale(inputs, ns)
        z128 = jnp.zeros((N, D_H), jnp.float32)
        z64 = jnp.zeros((N, D_OUT), jnp.float32)
        p1 = mp128(xs0, edge_index, z128)
        xs1 = layer(p1, nd, ns, W0, b0.reshape(1, -1))
        p2 = mp128(xs1, edge_index, z128)
        xs2 = layer2(p2, nd, ns, W1, b1.reshape(1, -1), W2)
        p3 = mp64(xs2, edge_index, z64)
        return final(p3, nd, b2.reshape(1, -1))

    return run


_cache = {}


def kernel(inputs, edge_index, W0, b0, W1, b1, W2, b2):
    if "run" not in _cache:
        _cache["run"] = _build()
    return _cache["run"](inputs, edge_index, W0, b0, W1, b1, W2, b2)


# speedup vs baseline: 13.4851x; 1.0001x over previous
import functools

import jax
import jax.numpy as jnp
from jax import lax
from jax.experimental import pallas as pl
from jax.experimental.pallas import tpu as pltpu
from jax.experimental.pallas import tpu_sc as plsc

N = 10000
E = 320000
D_IN = 128
D_H = 128
D_OUT = 64

NC = 2
NS = 16
NW = NC * NS
EPW = E // NW
CHUNK = 80
NCHUNK = EPW // CHUNK
RPT = 624
TAIL = N - NS * RPT

_SC_PARAMS = pltpu.CompilerParams(use_tc_tiling_on_sc=False,
                                  needs_layout_passes=False)

def _mesh():
    return plsc.VectorSubcoreMesh(core_axis_name="c", subcore_axis_name="s",
                                  num_cores=NC, num_subcores=NS)


def _build_deg(interpret=False):
    @functools.partial(
        pl.kernel,
        mesh=_mesh(),
        out_type=jax.ShapeDtypeStruct((2, NW, N), jnp.float32),
        scratch_types=[
            pltpu.VMEM((2, EPW), jnp.int32),
            pltpu.VMEM((N,), jnp.float32),
            pltpu.VMEM((N,), jnp.float32),
        ],
        compiler_params=_SC_PARAMS,
        interpret=interpret,
    )
    def deg_kernel(ei_hbm, degp_hbm, ebuf, hsrc, hdst):
        c = lax.axis_index("c")
        s = lax.axis_index("s")
        wid = c * NS + s
        pltpu.sync_copy(ei_hbm.at[:, pl.ds(wid * EPW, EPW)], ebuf)

        def zero(i, carry):
            hsrc[pl.ds(i * 16, 16)] = jnp.zeros((16,), jnp.float32)
            hdst[pl.ds(i * 16, 16)] = jnp.zeros((16,), jnp.float32)
            return carry

        lax.fori_loop(0, N // 16, zero, 0)

        ones = jnp.ones((16,), jnp.float32)

        def count(i, carry):
            sv = ebuf[0, pl.ds(i * 16, 16)]
            dv = ebuf[1, pl.ds(i * 16, 16)]
            plsc.addupdate_scatter(hsrc, [sv], ones)
            plsc.addupdate_scatter(hdst, [dv], ones)
            return carry

        lax.fori_loop(0, EPW // 16, count, 0)

        pltpu.sync_copy(hsrc, degp_hbm.at[0, wid])
        pltpu.sync_copy(hdst, degp_hbm.at[1, wid])

    return deg_kernel


def _build_mp(D, interpret=False):
    @functools.partial(
        pl.kernel,
        mesh=_mesh(),
        out_type=jax.ShapeDtypeStruct((NC, N, D), jnp.float32),
        scratch_types=[
            pltpu.VMEM((8, 2, CHUNK), jnp.int32),
            pltpu.VMEM((4, CHUNK, D), jnp.float32),
            pltpu.VMEM_SHARED((N, D), jnp.float32),
            pltpu.SemaphoreType.DMA((8,)),
            pltpu.SemaphoreType.DMA((4,)),
            pltpu.SemaphoreType.DMA((4,)),
        ],
        compiler_params=_SC_PARAMS,
        interpret=interpret,
    )
    def mp_kernel(xs_hbm, ei_hbm, z_hbm, out_hbm, ibuf, msgs, acc,
                  isem, gsem, ssem):
        c = lax.axis_index("c")
        s = lax.axis_index("s")
        wid = c * NS + s
        base = s * RPT

        ebase = wid * EPW

        def idx_load(j):
            b = j % 8
            return pltpu.async_copy(
                ei_hbm.at[:, pl.ds(ebase + j * CHUNK, CHUNK)],
                ibuf.at[b], isem.at[b])

        def gather(j):
            return pltpu.async_copy(xs_hbm.at[ibuf.at[j % 8, 0]],
                                    msgs.at[j % 4], gsem.at[j % 4])

        def scatter(j):
            return pltpu.async_copy(msgs.at[j % 4], acc.at[ibuf.at[j % 8, 1]],
                                    ssem.at[j % 4], add=True)

        idesc = {}
        gdesc = {}
        sdesc = {}
        for j in range(min(3, NCHUNK)):
            idesc[j] = idx_load(j)
        for j in range(min(2, NCHUNK)):
            idesc[j].wait()
            gdesc[j] = gather(j)

        pltpu.sync_copy(z_hbm.at[pl.ds(base, RPT)], acc.at[pl.ds(base, RPT)])

        @pl.when(s == NS - 1)
        def _zero_tail():
            pltpu.sync_copy(z_hbm.at[pl.ds(NS * RPT, TAIL)],
                            acc.at[pl.ds(NS * RPT, TAIL)])

        plsc.subcore_barrier()

        for j in range(NCHUNK):
            if j + 2 < NCHUNK and j >= 2:
                sdesc[j - 2].wait()
            if j + 3 < NCHUNK:
                idesc[j + 3] = idx_load(j + 3)
            if j + 2 < NCHUNK:
                idesc[j + 2].wait()
                gdesc[j + 2] = gather(j + 2)
            gdesc[j].wait()
            sdesc[j] = scatter(j)
        for k in range(max(0, NCHUNK - 4), NCHUNK):
            sdesc[k].wait()
        plsc.subcore_barrier()

        pltpu.sync_copy(acc.at[pl.ds(base, RPT)],
                        out_hbm.at[c, pl.ds(base, RPT)])

        @pl.when(s == NS - 1)
        def _copy_tail():
            pltpu.sync_copy(acc.at[pl.ds(NS * RPT, TAIL)],
                            out_hbm.at[c, pl.ds(NS * RPT, TAIL)])

    return mp_kernel


_BN = 2000


def _norm_body(degp_ref, x_ref, ns_ref, nd_ref, xs_ref):
    dsrc = jnp.sum(degp_ref[0], axis=1, keepdims=True)
    ddst = jnp.sum(degp_ref[1], axis=1, keepdims=True)
    ns = lax.rsqrt(jnp.maximum(dsrc, 1.0))
    nd = lax.rsqrt(jnp.maximum(ddst, 1.0))
    ns_ref[...] = ns
    nd_ref[...] = nd
    xs_ref[...] = x_ref[...] * ns


def _build_norm(interpret=False):
    return pl.pallas_call(
        _norm_body,
        grid=(N // _BN,),
        in_specs=[
            pl.BlockSpec((2, _BN, NW), lambda i: (0, i, 0)),
            pl.BlockSpec((_BN, D_IN), lambda i: (i, 0)),
        ],
        out_specs=[
            pl.BlockSpec((_BN, 1), lambda i: (i, 0)),
            pl.BlockSpec((_BN, 1), lambda i: (i, 0)),
            pl.BlockSpec((_BN, D_IN), lambda i: (i, 0)),
        ],
        out_shape=[
            jax.ShapeDtypeStruct((N, 1), jnp.float32),
            jax.ShapeDtypeStruct((N, 1), jnp.float32),
            jax.ShapeDtypeStruct((N, D_IN), jnp.float32),
        ],
        interpret=interpret,
    )


def _layer_body(p_ref, nd_ref, ns_ref, w_ref, b_ref, o_ref):
    agg = (p_ref[0] + p_ref[1]) * nd_ref[...]
    h = jnp.dot(agg, w_ref[...], preferred_element_type=jnp.float32,
                precision=lax.Precision.HIGHEST) + b_ref[...]
    o_ref[...] = jnp.maximum(h, 0.0) * ns_ref[...]


def _build_layer(interpret=False):
    return pl.pallas_call(
        _layer_body,
        grid=(N // _BN,),
        in_specs=[
            pl.BlockSpec((NC, _BN, D_H), lambda i: (0, i, 0)),
            pl.BlockSpec((_BN, 1), lambda i: (i, 0)),
            pl.BlockSpec((_BN, 1), lambda i: (i, 0)),
            pl.BlockSpec((D_H, D_H), lambda i: (0, 0)),
            pl.BlockSpec((1, D_H), lambda i: (0, 0)),
        ],
        out_specs=pl.BlockSpec((_BN, D_H), lambda i: (i, 0)),
        out_shape=jax.ShapeDtypeStruct((N, D_H), jnp.float32),
        interpret=interpret,
    )


def _layer2_body(p_ref, nd_ref, ns_ref, w1_ref, b1_ref, w2_ref, o_ref):
    agg = (p_ref[0] + p_ref[1]) * nd_ref[...]
    h = jnp.dot(agg, w1_ref[...], preferred_element_type=jnp.float32,
                precision=lax.Precision.HIGHEST) + b1_ref[...]
    h = jnp.maximum(h, 0.0) * ns_ref[...]
    o_ref[...] = jnp.dot(h, w2_ref[...], preferred_element_type=jnp.float32,
                         precision=lax.Precision.HIGHEST)


def _build_layer2(interpret=False):
    return pl.pallas_call(
        _layer2_body,
        grid=(N // _BN,),
        in_specs=[
            pl.BlockSpec((NC, _BN, D_H), lambda i: (0, i, 0)),
            pl.BlockSpec((_BN, 1), lambda i: (i, 0)),
            pl.BlockSpec((_BN, 1), lambda i: (i, 0)),
            pl.BlockSpec((D_H, D_H), lambda i: (0, 0)),
            pl.BlockSpec((1, D_H), lambda i: (0, 0)),
            pl.BlockSpec((D_H, D_OUT), lambda i: (0, 0)),
        ],
        out_specs=pl.BlockSpec((_BN, D_OUT), lambda i: (i, 0)),
        out_shape=jax.ShapeDtypeStruct((N, D_OUT), jnp.float32),
        interpret=interpret,
    )


def _final_body(p_ref, nd_ref, b_ref, o_ref):
    agg = (p_ref[0] + p_ref[1]) * nd_ref[...]
    o_ref[...] = jax.nn.sigmoid(agg + b_ref[...])


def _build_final(interpret=False):
    return pl.pallas_call(
        _final_body,
        grid=(N // _BN,),
        in_specs=[
            pl.BlockSpec((NC, _BN, D_OUT), lambda i: (0, i, 0)),
            pl.BlockSpec((_BN, 1), lambda i: (i, 0)),
            pl.BlockSpec((1, D_OUT), lambda i: (0, 0)),
        ],
        out_specs=pl.BlockSpec((_BN, D_OUT), lambda i: (i, 0)),
        out_shape=jax.ShapeDtypeStruct((N, D_OUT), jnp.float32),
        interpret=interpret,
    )


def _build(interpret=False):
    deg = _build_deg(interpret)
    mp128 = _build_mp(D_H, interpret)
    mp64 = _build_mp(D_OUT, interpret)
    norm = _build_norm(interpret)
    layer = _build_layer(interpret)
    layer2 = _build_layer2(interpret)
    final = _build_final(interpret)

    def run(inputs, edge_index, W0, b0, W1, b1, W2, b2):
        degp = deg(edge_index)
        degp_t = jnp.transpose(degp, (0, 2, 1))
        ns, nd, xs0 = norm(degp_t, inputs)
        z128 = jnp.zeros((N, D_H), jnp.float32)
        z64 = jnp.zeros((N, D_OUT), jnp.float32)
        p1 = mp128(xs0, edge_index, z128)
        xs1 = layer(p1, nd, ns, W0, b0.reshape(1, -1))
        p2 = mp128(xs1, edge_index, z128)
        xs2 = layer2(p2, nd, ns, W1, b1.reshape(1, -1), W2)
        p3 = mp64(xs2, edge_index, z64)
        return final(p3, nd, b2.reshape(1, -1))

    return run


_cache = {}


def kernel(inputs, edge_index, W0, b0, W1, b1, W2, b2):
    if "run" not in _cache:
        _cache["run"] = _build()
    return _cache["run"](inputs, edge_index, W0, b0, W1, b1, W2, b2)


# speedup vs baseline: 13.8738x; 1.0288x over previous
import functools

import jax
import jax.numpy as jnp
from jax import lax
from jax.experimental import pallas as pl
from jax.experimental.pallas import tpu as pltpu
from jax.experimental.pallas import tpu_sc as plsc

N = 10000
E = 320000
D_IN = 128
D_H = 128
D_OUT = 64

NC = 2
NS = 16
NW = NC * NS
EPW = E // NW
CHUNK = 128
NFULL = EPW // CHUNK
TAILE = EPW - NFULL * CHUNK
RPT = 624
TAIL = N - NS * RPT

_SC_PARAMS = pltpu.CompilerParams(use_tc_tiling_on_sc=False,
                                  needs_layout_passes=False)

def _mesh():
    return plsc.VectorSubcoreMesh(core_axis_name="c", subcore_axis_name="s",
                                  num_cores=NC, num_subcores=NS)


def _build_deg():
    @functools.partial(
        pl.kernel,
        mesh=_mesh(),
        out_type=jax.ShapeDtypeStruct((2, NW, N), jnp.float32),
        scratch_types=[
            pltpu.VMEM((2, EPW), jnp.int32),
            pltpu.VMEM((N,), jnp.float32),
            pltpu.VMEM((N,), jnp.float32),
        ],
        compiler_params=_SC_PARAMS,
    )
    def deg_kernel(ei_hbm, degp_hbm, ebuf, hsrc, hdst):
        c = lax.axis_index("c")
        s = lax.axis_index("s")
        wid = c * NS + s
        pltpu.sync_copy(ei_hbm.at[:, pl.ds(wid * EPW, EPW)], ebuf)

        def zero(i, carry):
            hsrc[pl.ds(i * 16, 16)] = jnp.zeros((16,), jnp.float32)
            hdst[pl.ds(i * 16, 16)] = jnp.zeros((16,), jnp.float32)
            return carry

        lax.fori_loop(0, N // 16, zero, 0)

        ones = jnp.ones((16,), jnp.float32)

        def count(i, carry):
            sv = ebuf[0, pl.ds(i * 16, 16)]
            dv = ebuf[1, pl.ds(i * 16, 16)]
            plsc.addupdate_scatter(hsrc, [sv], ones)
            plsc.addupdate_scatter(hdst, [dv], ones)
            return carry

        lax.fori_loop(0, EPW // 16, count, 0)

        pltpu.sync_copy(hsrc, degp_hbm.at[0, wid])
        pltpu.sync_copy(hdst, degp_hbm.at[1, wid])

    return deg_kernel


def _build_mp(D):
    MD = 3 if D == D_H else 6
    ID = 2 * MD

    @functools.partial(
        pl.kernel,
        mesh=_mesh(),
        out_type=jax.ShapeDtypeStruct((NC, N, D), jnp.float32),
        scratch_types=[
            pltpu.VMEM((ID, 2, CHUNK), jnp.int32),
            pltpu.VMEM((2, TAILE), jnp.int32),
            pltpu.VMEM((MD, CHUNK, D), jnp.float32),
            pltpu.VMEM_SHARED((N, D), jnp.float32),
            pltpu.SemaphoreType.DMA((ID,)),
            pltpu.SemaphoreType.DMA((MD,)),
            pltpu.SemaphoreType.DMA((MD,)),
        ],
        compiler_params=_SC_PARAMS,
    )
    def mp_kernel(xs_hbm, ei_hbm, z_hbm, out_hbm, ibuf, tibuf, msgs, acc,
                  isem, gsem, ssem):
        c = lax.axis_index("c")
        s = lax.axis_index("s")
        wid = c * NS + s
        base = s * RPT
        ebase = wid * EPW

        def idx_load(j):
            b = j % ID
            return pltpu.async_copy(
                ei_hbm.at[:, pl.ds(ebase + j * CHUNK, CHUNK)],
                ibuf.at[b], isem.at[b])

        def gather(j):
            return pltpu.async_copy(xs_hbm.at[ibuf.at[j % ID, 0]],
                                    msgs.at[j % MD], gsem.at[j % MD])

        def scatter(j):
            return pltpu.async_copy(msgs.at[j % MD], acc.at[ibuf.at[j % ID, 1]],
                                    ssem.at[j % MD], add=True)

        idesc = {}
        gdesc = {}
        sdesc = {}
        idesc[0] = idx_load(0)
        idesc[1] = idx_load(1)
        idesc[0].wait()
        gdesc[0] = gather(0)

        pltpu.sync_copy(z_hbm.at[pl.ds(base, RPT)], acc.at[pl.ds(base, RPT)])

        @pl.when(s == NS - 1)
        def _zero_tail():
            pltpu.sync_copy(z_hbm.at[pl.ds(NS * RPT, TAIL)],
                            acc.at[pl.ds(NS * RPT, TAIL)])

        plsc.subcore_barrier()

        for j in range(NFULL):
            if j >= MD - 1:
                sdesc[j - (MD - 1)].wait()
            if j + 2 < NFULL:
                idesc[j + 2] = idx_load(j + 2)
            if j + 1 < NFULL:
                idesc[j + 1].wait()
                gdesc[j + 1] = gather(j + 1)
            gdesc[j].wait()
            sdesc[j] = scatter(j)
        for k in range(max(0, NFULL - (MD - 1)), NFULL):
            sdesc[k].wait()

        pltpu.sync_copy(ei_hbm.at[:, pl.ds(ebase + NFULL * CHUNK, TAILE)],
                        tibuf)
        pltpu.async_copy(xs_hbm.at[tibuf.at[0]],
                         msgs.at[0, pl.ds(0, TAILE)], gsem.at[0]).wait()
        pltpu.async_copy(msgs.at[0, pl.ds(0, TAILE)],
                         acc.at[tibuf.at[1]], ssem.at[0], add=True).wait()
        plsc.subcore_barrier()

        pltpu.sync_copy(acc.at[pl.ds(base, RPT)],
                        out_hbm.at[c, pl.ds(base, RPT)])

        @pl.when(s == NS - 1)
        def _copy_tail():
            pltpu.sync_copy(acc.at[pl.ds(NS * RPT, TAIL)],
                            out_hbm.at[c, pl.ds(NS * RPT, TAIL)])

    return mp_kernel


_BN = 2000


def _norm_body(degp_ref, x_ref, ns_ref, nd_ref, xs_ref):
    dsrc = jnp.sum(degp_ref[0], axis=1, keepdims=True)
    ddst = jnp.sum(degp_ref[1], axis=1, keepdims=True)
    ns = lax.rsqrt(jnp.maximum(dsrc, 1.0))
    nd = lax.rsqrt(jnp.maximum(ddst, 1.0))
    ns_ref[...] = ns
    nd_ref[...] = nd
    xs_ref[...] = x_ref[...] * ns


def _build_norm():
    return pl.pallas_call(
        _norm_body,
        grid=(N // _BN,),
        in_specs=[
            pl.BlockSpec((2, _BN, NW), lambda i: (0, i, 0)),
            pl.BlockSpec((_BN, D_IN), lambda i: (i, 0)),
        ],
        out_specs=[
            pl.BlockSpec((_BN, 1), lambda i: (i, 0)),
            pl.BlockSpec((_BN, 1), lambda i: (i, 0)),
            pl.BlockSpec((_BN, D_IN), lambda i: (i, 0)),
        ],
        out_shape=[
            jax.ShapeDtypeStruct((N, 1), jnp.float32),
            jax.ShapeDtypeStruct((N, 1), jnp.float32),
            jax.ShapeDtypeStruct((N, D_IN), jnp.float32),
        ],
    )


def _mm0_body(x_ref, w_ref, o_ref):
    o_ref[...] = jnp.dot(x_ref[...], w_ref[...],
                         preferred_element_type=jnp.float32,
                         precision=lax.Precision.HIGHEST)


def _build_mm0():
    return pl.pallas_call(
        _mm0_body,
        grid=(N // _BN,),
        in_specs=[
            pl.BlockSpec((_BN, D_IN), lambda i: (i, 0)),
            pl.BlockSpec((D_IN, D_H), lambda i: (0, 0)),
        ],
        out_specs=pl.BlockSpec((_BN, D_H), lambda i: (i, 0)),
        out_shape=jax.ShapeDtypeStruct((N, D_H), jnp.float32),
    )


def _layer_body(p_ref, nd_ref, ns_ref, b_ref, w_ref, o_ref):
    agg = (p_ref[0] + p_ref[1]) * nd_ref[...]
    h = jnp.maximum(agg + b_ref[...], 0.0) * ns_ref[...]
    o_ref[...] = jnp.dot(h, w_ref[...], preferred_element_type=jnp.float32,
                         precision=lax.Precision.HIGHEST)


def _build_layer(dout):
    return pl.pallas_call(
        _layer_body,
        grid=(N // _BN,),
        in_specs=[
            pl.BlockSpec((NC, _BN, D_H), lambda i: (0, i, 0)),
            pl.BlockSpec((_BN, 1), lambda i: (i, 0)),
            pl.BlockSpec((_BN, 1), lambda i: (i, 0)),
            pl.BlockSpec((1, D_H), lambda i: (0, 0)),
            pl.BlockSpec((D_H, dout), lambda i: (0, 0)),
        ],
        out_specs=pl.BlockSpec((_BN, dout), lambda i: (i, 0)),
        out_shape=jax.ShapeDtypeStruct((N, dout), jnp.float32),
    )


def _final_body(p_ref, nd_ref, b_ref, o_ref):
    agg = (p_ref[0] + p_ref[1]) * nd_ref[...]
    o_ref[...] = jax.nn.sigmoid(agg + b_ref[...])


def _build_final():
    return pl.pallas_call(
        _final_body,
        grid=(N // _BN,),
        in_specs=[
            pl.BlockSpec((NC, _BN, D_OUT), lambda i: (0, i, 0)),
            pl.BlockSpec((_BN, 1), lambda i: (i, 0)),
            pl.BlockSpec((1, D_OUT), lambda i: (0, 0)),
        ],
        out_specs=pl.BlockSpec((_BN, D_OUT), lambda i: (i, 0)),
        out_shape=jax.ShapeDtypeStruct((N, D_OUT), jnp.float32),
    )


def _build():
    deg = _build_deg()
    mp128 = _build_mp(D_H)
    mp64 = _build_mp(D_OUT)
    mm0 = _build_mm0()
    norm = _build_norm()
    layer128 = _build_layer(D_H)
    layer64 = _build_layer(D_OUT)
    final = _build_final()

    def run(inputs, edge_index, W0, b0, W1, b1, W2, b2):
        y0 = mm0(inputs, W0)
        degp = deg(edge_index)
        degp_t = jnp.transpose(degp, (0, 2, 1))
        ns, nd, xs0 = norm(degp_t, y0)
        z128 = jnp.zeros((N, D_H), jnp.float32)
        z64 = jnp.zeros((N, D_OUT), jnp.float32)
        p1 = mp128(xs0, edge_index, z128)
        xs1 = layer128(p1, nd, ns, b0.reshape(1, -1), W1)
        p2 = mp128(xs1, edge_index, z128)
        xs2 = layer64(p2, nd, ns, b1.reshape(1, -1), W2)
        p3 = mp64(xs2, edge_index, z64)
        return final(p3, nd, b2.reshape(1, -1))

    return run


_cache = {}


def kernel(inputs, edge_index, W0, b0, W1, b1, W2, b2):
    if "run" not in _cache:
        _cache["run"] = _build()
    return _cache["run"](inputs, edge_index, W0, b0, W1, b1, W2, b2)
